# baseline (speedup 1.0000x reference)
import functools

import jax
import jax.numpy as jnp
from jax import lax
from jax.experimental import pallas as pl
from jax.experimental.pallas import tpu as pltpu
from jax.experimental.pallas import tpu_sc as plsc

NC = 2
NS = 16
NW = NC * NS

CH = 128
N_CHUNK = 80
N_HALF = N_CHUNK // 2
ZR = 32
PAD_N = 10240
DUMP_ROW = 10100


def _sc_segment_sum(table, gidx3, sidx3):
  d = table.shape[1]
  rows_pt = PAD_N // NS
  mesh = plsc.VectorSubcoreMesh(core_axis_name="c", subcore_axis_name="s")

  @functools.partial(
      pl.kernel,
      out_type=jax.ShapeDtypeStruct((NC, PAD_N, d), jnp.float32),
      mesh=mesh,
      scratch_types=[
          pltpu.VMEM((N_HALF, CH), jnp.int32),
          pltpu.VMEM((N_HALF, CH), jnp.int32),
          pltpu.VMEM((CH, d), jnp.float32),
          pltpu.VMEM((CH, d), jnp.float32),
          pltpu.VMEM((ZR, d), jnp.float32),
          pltpu.VMEM_SHARED((PAD_N, d), jnp.float32),
          pltpu.SemaphoreType.DMA,
          pltpu.SemaphoreType.DMA,
          pltpu.SemaphoreType.DMA,
          pltpu.SemaphoreType.DMA,
      ],
  )
  def k(tbl_hbm, gidx_hbm, sidx_hbm, out_hbm, gv, sv, rows0, rows1, zbuf,
        acc, sem0, sem1, ssem0, ssem1):
    c = lax.axis_index("c")
    s = lax.axis_index("s")
    wid = c * NS + s

    pltpu.async_copy(gidx_hbm.at[wid, pl.ds(0, N_HALF)], gv, sem1)
    pltpu.async_copy(sidx_hbm.at[wid, pl.ds(0, N_HALF)], sv, sem1)

    zero16 = jnp.zeros((16,), jnp.float32)

    def zrow(i, _):
      def zcol(j, _):
        zbuf[i, pl.ds(j * 16, 16)] = zero16
        return 0
      return lax.fori_loop(0, d // 16, zcol, 0)

    lax.fori_loop(0, ZR, zrow, 0)

    def zcopy(r, _):
      pltpu.async_copy(zbuf, acc.at[pl.ds(s * rows_pt + r * ZR, ZR)], sem0)
      return 0

    lax.fori_loop(0, rows_pt // ZR, zcopy, 0)

    def zwait(r, _):
      pltpu.make_async_copy(
          zbuf, acc.at[pl.ds(s * rows_pt + r * ZR, ZR)], sem0).wait()
      return 0

    lax.fori_loop(0, rows_pt // ZR, zwait, 0)
    plsc.subcore_barrier()

    n_t = N_HALF // 2
    for half in range(2):
      if half == 0:
        pltpu.make_async_copy(gidx_hbm.at[wid, pl.ds(0, N_HALF)], gv,
                              sem1).wait()
        pltpu.make_async_copy(sidx_hbm.at[wid, pl.ds(0, N_HALF)], sv,
                              sem1).wait()
      else:
        pltpu.sync_copy(gidx_hbm.at[wid, pl.ds(N_HALF, N_HALF)], gv)
        pltpu.sync_copy(sidx_hbm.at[wid, pl.ds(N_HALF, N_HALF)], sv)
      pltpu.async_copy(tbl_hbm.at[gv.at[0]], rows0, sem0)
      pltpu.async_copy(tbl_hbm.at[gv.at[1]], rows1, sem1)

      def body(t, _):
        j = 2 * t
        pltpu.make_async_copy(tbl_hbm.at[gv.at[j]], rows0, sem0).wait()
        pltpu.async_copy(rows0, acc.at[sv.at[j]], ssem0, add=True)
        pltpu.make_async_copy(tbl_hbm.at[gv.at[j + 1]], rows1, sem1).wait()
        pltpu.async_copy(rows1, acc.at[sv.at[j + 1]], ssem1, add=True)

        @pl.when(t + 1 < n_t)
        def _():
          pltpu.make_async_copy(rows0, acc.at[sv.at[j]], ssem0).wait()
          pltpu.async_copy(tbl_hbm.at[gv.at[j + 2]], rows0, sem0)
          pltpu.make_async_copy(rows1, acc.at[sv.at[j + 1]], ssem1).wait()
          pltpu.async_copy(tbl_hbm.at[gv.at[j + 3]], rows1, sem1)
        return 0

      lax.fori_loop(0, n_t, body, 0)
      pltpu.make_async_copy(rows0, acc.at[sv.at[N_HALF - 2]], ssem0).wait()
      pltpu.make_async_copy(rows1, acc.at[sv.at[N_HALF - 1]], ssem1).wait()
    plsc.subcore_barrier()

    n_dr = rows_pt // ZR

    def wcopy(r, _):
      rs = s * rows_pt + r * ZR
      pltpu.async_copy(acc.at[pl.ds(rs, ZR)], out_hbm.at[c, pl.ds(rs, ZR)],
                       sem0)
      return 0

    lax.fori_loop(0, n_dr, wcopy, 0)

    def wdrain(r, _):
      rs = s * rows_pt + r * ZR
      pltpu.make_async_copy(acc.at[pl.ds(rs, ZR)],
                            out_hbm.at[c, pl.ds(rs, ZR)], sem0).wait()
      return 0

    lax.fori_loop(0, n_dr, wdrain, 0)

  return k(table, gidx3, sidx3)


def _combine(p, n):
  d = p.shape[2]
  bm = 1000

  def body(p_ref, o_ref):
    o_ref[...] = p_ref[0] + p_ref[1]

  return pl.pallas_call(
      body,
      grid=(n // bm,),
      in_specs=[pl.BlockSpec((NC, bm, d), lambda i: (0, i, 0))],
      out_specs=pl.BlockSpec((bm, d), lambda i: (i, 0)),
      out_shape=jax.ShapeDtypeStruct((n, d), jnp.float32),
  )(p)


def _gin_update(x0, q, w, b2, eps2):
  n, d = x0.shape
  bm = 1000

  def body(eps_ref, x_ref, q_ref, w_ref, b_ref, o_ref):
    scale = 1.0 + eps_ref[0, 0]
    a = x_ref[...] * scale + q_ref[0] + q_ref[1]
    o_ref[...] = lax.dot_general(
        a, w_ref[...], (((1,), (1,)), ((), ())),
        preferred_element_type=jnp.float32) + b_ref[...]

  return pl.pallas_call(
      body,
      grid=(n // bm,),
      in_specs=[
          pl.BlockSpec(memory_space=pltpu.SMEM),
          pl.BlockSpec((bm, d), lambda i: (i, 0)),
          pl.BlockSpec((NC, bm, d), lambda i: (0, i, 0)),
          pl.BlockSpec((d, d), lambda i: (0, 0)),
          pl.BlockSpec((1, d), lambda i: (0, 0)),
      ],
      out_specs=pl.BlockSpec((bm, d), lambda i: (i, 0)),
      out_shape=jax.ShapeDtypeStruct((n, d), jnp.float32),
  )(eps2, x0, q, w, b2)


def kernel(x_0, incidence_indices, W, b, eps):
  n_nodes, d = x_0.shape
  nnz = incidence_indices.shape[1]
  n_hedges = n_nodes
  nnz_p = NW * N_CHUNK * CH
  pad = nnz_p - nnz

  node_idx = incidence_indices[0]
  edge_idx = incidence_indices[1]
  pad_g = (jnp.arange(pad, dtype=jnp.int32) * 37) % n_nodes
  pad_s = DUMP_ROW + (jnp.arange(pad, dtype=jnp.int32) % (PAD_N - DUMP_ROW))
  shape3 = (NW, N_CHUNK, CH)
  nidx_g = jnp.concatenate([node_idx, pad_g]).reshape(shape3)
  nidx_s = jnp.concatenate([node_idx, pad_s]).reshape(shape3)
  eidx_g = jnp.concatenate([edge_idx, pad_g]).reshape(shape3)
  eidx_s = jnp.concatenate([edge_idx, pad_s]).reshape(shape3)

  p = _sc_segment_sum(x_0, nidx_g, eidx_s)
  x_1 = _combine(p, n_hedges)
  q = _sc_segment_sum(x_1, eidx_g, nidx_s)
  x0_out = _gin_update(x_0, q, W, b.reshape(1, d), eps.reshape(1, 1))
  return (x0_out, x_1)


# speedup vs baseline: 1.2772x; 1.2772x over previous
import functools

import jax
import jax.numpy as jnp
from jax import lax
from jax.experimental import pallas as pl
from jax.experimental.pallas import tpu as pltpu
from jax.experimental.pallas import tpu_sc as plsc

NC = 2
NS = 16
NW = NC * NS

CH = 128
N_CHUNK = 80
N_HALF = N_CHUNK // 2
ZR = 32
PAD_N = 10240
DUMP_ROW = 10100


def _sc_segment_sum(table, gidx3, sidx3):
  d = table.shape[1]
  rows_pt = PAD_N // NS
  mesh = plsc.VectorSubcoreMesh(core_axis_name="c", subcore_axis_name="s")

  @functools.partial(
      pl.kernel,
      out_type=jax.ShapeDtypeStruct((NC, PAD_N, d), jnp.float32),
      mesh=mesh,
      scratch_types=[
          pltpu.VMEM((N_HALF, CH), jnp.int32),
          pltpu.VMEM((N_HALF, CH), jnp.int32),
          pltpu.VMEM((CH, d), jnp.float32),
          pltpu.VMEM((CH, d), jnp.float32),
          pltpu.VMEM((ZR, d), jnp.float32),
          pltpu.VMEM_SHARED((PAD_N, d), jnp.float32),
          pltpu.SemaphoreType.DMA,
          pltpu.SemaphoreType.DMA,
      ],
  )
  def k(tbl_hbm, gidx_hbm, sidx_hbm, out_hbm, gv, sv, rows0, rows1, zbuf,
        acc, sem0, sem1):
    c = lax.axis_index("c")
    s = lax.axis_index("s")
    wid = c * NS + s

    pltpu.async_copy(gidx_hbm.at[wid, pl.ds(0, N_HALF)], gv, sem1)
    pltpu.async_copy(sidx_hbm.at[wid, pl.ds(0, N_HALF)], sv, sem1)

    zero16 = jnp.zeros((16,), jnp.float32)

    def zrow(i, _):
      def zcol(j, _):
        zbuf[i, pl.ds(j * 16, 16)] = zero16
        return 0
      return lax.fori_loop(0, d // 16, zcol, 0)

    lax.fori_loop(0, ZR, zrow, 0)

    def zcopy(r, _):
      pltpu.async_copy(zbuf, acc.at[pl.ds(s * rows_pt + r * ZR, ZR)], sem0)
      return 0

    lax.fori_loop(0, rows_pt // ZR, zcopy, 0)

    def zwait(r, _):
      pltpu.make_async_copy(
          zbuf, acc.at[pl.ds(s * rows_pt + r * ZR, ZR)], sem0).wait()
      return 0

    lax.fori_loop(0, rows_pt // ZR, zwait, 0)
    plsc.subcore_barrier()

    n_t = N_HALF // 2
    for half in range(2):
      if half == 0:
        pltpu.make_async_copy(gidx_hbm.at[wid, pl.ds(0, N_HALF)], gv,
                              sem1).wait()
        pltpu.make_async_copy(sidx_hbm.at[wid, pl.ds(0, N_HALF)], sv,
                              sem1).wait()
      else:
        pltpu.sync_copy(gidx_hbm.at[wid, pl.ds(N_HALF, N_HALF)], gv)
        pltpu.sync_copy(sidx_hbm.at[wid, pl.ds(N_HALF, N_HALF)], sv)
      pltpu.async_copy(tbl_hbm.at[gv.at[0]], rows0, sem0)

      def body(t, _):
        j = 2 * t
        pltpu.async_copy(tbl_hbm.at[gv.at[j + 1]], rows1, sem1)
        pltpu.make_async_copy(tbl_hbm.at[gv.at[j]], rows0, sem0).wait()
        pltpu.sync_copy(rows0, acc.at[sv.at[j]], add=True)

        @pl.when(t + 1 < n_t)
        def _():
          pltpu.async_copy(tbl_hbm.at[gv.at[j + 2]], rows0, sem0)

        pltpu.make_async_copy(tbl_hbm.at[gv.at[j + 1]], rows1, sem1).wait()
        pltpu.sync_copy(rows1, acc.at[sv.at[j + 1]], add=True)
        return 0

      lax.fori_loop(0, n_t, body, 0)
    plsc.subcore_barrier()

    n_dr = rows_pt // ZR

    def wcopy(r, _):
      rs = s * rows_pt + r * ZR
      pltpu.async_copy(acc.at[pl.ds(rs, ZR)], out_hbm.at[c, pl.ds(rs, ZR)],
                       sem0)
      return 0

    lax.fori_loop(0, n_dr, wcopy, 0)

    def wdrain(r, _):
      rs = s * rows_pt + r * ZR
      pltpu.make_async_copy(acc.at[pl.ds(rs, ZR)],
                            out_hbm.at[c, pl.ds(rs, ZR)], sem0).wait()
      return 0

    lax.fori_loop(0, n_dr, wdrain, 0)

  return k(table, gidx3, sidx3)


def _combine(p, n):
  d = p.shape[2]
  bm = 1000

  def body(p_ref, o_ref):
    o_ref[...] = p_ref[0] + p_ref[1]

  return pl.pallas_call(
      body,
      grid=(n // bm,),
      in_specs=[pl.BlockSpec((NC, bm, d), lambda i: (0, i, 0))],
      out_specs=pl.BlockSpec((bm, d), lambda i: (i, 0)),
      out_shape=jax.ShapeDtypeStruct((n, d), jnp.float32),
  )(p)


def _gin_update(x0, q, w, b2, eps2):
  n, d = x0.shape
  bm = 1000

  def body(eps_ref, x_ref, q_ref, w_ref, b_ref, o_ref):
    scale = 1.0 + eps_ref[0, 0]
    a = x_ref[...] * scale + q_ref[0] + q_ref[1]
    o_ref[...] = lax.dot_general(
        a, w_ref[...], (((1,), (1,)), ((), ())),
        preferred_element_type=jnp.float32) + b_ref[...]

  return pl.pallas_call(
      body,
      grid=(n // bm,),
      in_specs=[
          pl.BlockSpec(memory_space=pltpu.SMEM),
          pl.BlockSpec((bm, d), lambda i: (i, 0)),
          pl.BlockSpec((NC, bm, d), lambda i: (0, i, 0)),
          pl.BlockSpec((d, d), lambda i: (0, 0)),
          pl.BlockSpec((1, d), lambda i: (0, 0)),
      ],
      out_specs=pl.BlockSpec((bm, d), lambda i: (i, 0)),
      out_shape=jax.ShapeDtypeStruct((n, d), jnp.float32),
  )(eps2, x0, q, w, b2)


def kernel(x_0, incidence_indices, W, b, eps):
  n_nodes, d = x_0.shape
  nnz = incidence_indices.shape[1]
  n_hedges = n_nodes
  nnz_p = NW * N_CHUNK * CH
  pad = nnz_p - nnz

  node_idx = incidence_indices[0]
  edge_idx = incidence_indices[1]
  pad_g = (jnp.arange(pad, dtype=jnp.int32) * 37) % n_nodes
  pad_s = DUMP_ROW + (jnp.arange(pad, dtype=jnp.int32) % (PAD_N - DUMP_ROW))
  shape3 = (NW, N_CHUNK, CH)
  nidx_g = jnp.concatenate([node_idx, pad_g]).reshape(shape3)
  nidx_s = jnp.concatenate([node_idx, pad_s]).reshape(shape3)
  eidx_g = jnp.concatenate([edge_idx, pad_g]).reshape(shape3)
  eidx_s = jnp.concatenate([edge_idx, pad_s]).reshape(shape3)

  p = _sc_segment_sum(x_0, nidx_g, eidx_s)
  x_1 = _combine(p, n_hedges)
  q = _sc_segment_sum(x_1, eidx_g, nidx_s)
  x0_out = _gin_update(x_0, q, W, b.reshape(1, d), eps.reshape(1, 1))
  return (x0_out, x_1)
